# baseline (speedup 1.0000x reference)
import functools

import jax
import jax.numpy as jnp
from jax import lax
from jax.experimental import pallas as pl
from jax.experimental.pallas import tpu as pltpu
from jax.experimental.pallas import tpu_sc as plsc

_B = 16384
_EMB = 64
_NC = 2
_NS = 16
_NW = _NC * _NS
_BPW = _B // _NW


def _make_sc_gather():
    mesh = plsc.VectorSubcoreMesh(core_axis_name="c", subcore_axis_name="s")

    @functools.partial(
        pl.kernel,
        mesh=mesh,
        compiler_params=pltpu.CompilerParams(use_tc_tiling_on_sc=False),
        out_type=[
            jax.ShapeDtypeStruct((_B, _EMB), jnp.float32),
            jax.ShapeDtypeStruct((_B, _EMB), jnp.float32),
        ],
        scratch_types=[
            pltpu.VMEM((_BPW,), jnp.int32),
            pltpu.VMEM((_BPW, _EMB), jnp.float32),
            pltpu.VMEM((_BPW,), jnp.int32),
            pltpu.VMEM((_BPW, _EMB), jnp.float32),
            pltpu.SemaphoreType.DMA,
            pltpu.SemaphoreType.DMA,
        ],
    )
    def gather2(uidx_hbm, utab_hbm, gidx_hbm, gtab_hbm, uout_hbm, gout_hbm,
                uidx_v, urows_v, gidx_v, grows_v, usem, gsem):
        wid = lax.axis_index("s") * _NC + lax.axis_index("c")
        base = wid * _BPW
        pltpu.sync_copy(uidx_hbm.at[pl.ds(base, _BPW)], uidx_v)
        pltpu.sync_copy(gidx_hbm.at[pl.ds(base, _BPW)], gidx_v)
        cu = pltpu.async_copy(utab_hbm.at[uidx_v], urows_v, usem)
        cg = pltpu.async_copy(gtab_hbm.at[gidx_v], grows_v, gsem)
        cu.wait()
        cg.wait()
        pltpu.sync_copy(urows_v, uout_hbm.at[pl.ds(base, _BPW)])
        pltpu.sync_copy(grows_v, gout_hbm.at[pl.ds(base, _BPW)])

    return gather2


_sc_gather_cache = []


def _sc_gather(uidx, utab, gidx, gtab):
    if not _sc_gather_cache:
        _sc_gather_cache.append(_make_sc_gather())
    return _sc_gather_cache[0](uidx, utab, gidx, gtab)


def _mlp_body(gf_ref, gl_ref, umf_ref, gmf_ref,
              gw1_ref, gb1_ref, gw2_ref, gb2_ref,
              uw1_ref, ub1_ref, uw2_ref, ub2_ref,
              dw1a_ref, dw1b_ref, db1_ref, dw2_ref, db2_ref,
              out_ref):
    f32 = jnp.float32
    g1 = jnp.maximum(
        jnp.dot(gf_ref[...], gw1_ref[...], preferred_element_type=f32) + gb1_ref[...], 0.0)
    genc = jnp.maximum(
        jnp.dot(g1, gw2_ref[...], preferred_element_type=f32) + gb2_ref[...], 0.0)
    u1 = jnp.maximum(
        jnp.dot(gl_ref[...], uw1_ref[...], preferred_element_type=f32) + ub1_ref[...], 0.0)
    uenc = jnp.maximum(
        jnp.dot(u1, uw2_ref[...], preferred_element_type=f32) + ub2_ref[...], 0.0)
    fu = umf_ref[...] + uenc
    fg = gmf_ref[...] + genc
    h = jnp.maximum(
        jnp.dot(fu, dw1a_ref[...], preferred_element_type=f32)
        + jnp.dot(fg, dw1b_ref[...], preferred_element_type=f32)
        + db1_ref[...], 0.0)
    out_ref[...] = jnp.dot(h, dw2_ref[...], preferred_element_type=f32) + db2_ref[...]


_R = 2048


def _dense(gf, gl, umf, gmf, gW1, gb1, gW2, gb2, uW1, ub1, uW2, ub2,
           dW1a, dW1b, db1, dW2, db2):
    nblk = _B // _R

    def rows(i):
        return (i, 0)

    def whole(i):
        return (0, 0)

    row_spec_feat = pl.BlockSpec((_R, gf.shape[1]), rows)
    row_spec_emb = pl.BlockSpec((_R, _EMB), rows)

    def wspec(a):
        return pl.BlockSpec(a.shape, whole)

    out = pl.pallas_call(
        _mlp_body,
        grid=(nblk,),
        in_specs=[
            row_spec_feat, row_spec_feat, row_spec_emb, row_spec_emb,
            wspec(gW1), wspec(gb1), wspec(gW2), wspec(gb2),
            wspec(uW1), wspec(ub1), wspec(uW2), wspec(ub2),
            wspec(dW1a), wspec(dW1b), wspec(db1), wspec(dW2), wspec(db2),
        ],
        out_specs=pl.BlockSpec((_R, 1), rows),
        out_shape=jax.ShapeDtypeStruct((_B, 1), jnp.float32),
    )(gf, gl, umf, gmf, gW1, gb1, gW2, gb2, uW1, ub1, uW2, ub2,
      dW1a, dW1b, db1, dW2, db2)
    return out[:, 0]


def kernel(user_input, game_input, game_features, global_features,
           user_table, game_table,
           gW1, gb1, gW2, gb2,
           uW1, ub1, uW2, ub2,
           dW1, db1, dW2, db2):
    umf, gmf = _sc_gather(user_input, user_table, game_input, game_table)
    dW1a = dW1[:_EMB]
    dW1b = dW1[_EMB:]
    return _dense(
        game_features, global_features, umf, gmf,
        gW1, gb1.reshape(1, -1), gW2, gb2.reshape(1, -1),
        uW1, ub1.reshape(1, -1), uW2, ub2.reshape(1, -1),
        dW1a, dW1b, db1.reshape(1, -1), dW2, db2.reshape(1, -1))


# speedup vs baseline: 1.5620x; 1.5620x over previous
import functools

import jax
import jax.numpy as jnp
from jax import lax
from jax.experimental import pallas as pl
from jax.experimental.pallas import tpu as pltpu
from jax.experimental.pallas import tpu_sc as plsc

_B = 16384
_EMB = 64
_NC = 2
_NS = 16
_NW = _NC * _NS
_BPW = _B // _NW


_K = 16


def _make_sc_gather():
    mesh = plsc.VectorSubcoreMesh(core_axis_name="c", subcore_axis_name="s")

    @functools.partial(
        pl.kernel,
        mesh=mesh,
        compiler_params=pltpu.CompilerParams(needs_layout_passes=False),
        out_type=[
            jax.ShapeDtypeStruct((_B, _EMB), jnp.float32),
            jax.ShapeDtypeStruct((_B, _EMB), jnp.float32),
        ],
        scratch_types=[
            pltpu.SMEM((_BPW,), jnp.int32),
            pltpu.VMEM((_BPW // 2, _EMB), jnp.float32),
            pltpu.SMEM((_BPW,), jnp.int32),
            pltpu.VMEM((_BPW // 2, _EMB), jnp.float32),
            pltpu.VMEM((_BPW,), jnp.int32),
            pltpu.VMEM((_BPW,), jnp.int32),
            pltpu.SemaphoreType.DMA,
            pltpu.SemaphoreType.DMA,
        ],
    )
    def gather2(uidx_hbm, utab_hbm, gidx_hbm, gtab_hbm, uout_hbm, gout_hbm,
                uidx_s, urows_v, gidx_s, grows_v, uidx_v, gidx_v, usem, gsem):
        wid = lax.axis_index("s") * _NC + lax.axis_index("c")
        base = wid * _BPW
        half = _BPW // 2
        pltpu.sync_copy(uidx_hbm.at[pl.ds(base, _BPW)], uidx_v)
        pltpu.sync_copy(gidx_hbm.at[pl.ds(base, _BPW)], gidx_v)

        lanes = lax.iota(jnp.int32, 16)

        def to_smem(g, _):
            uv = uidx_v[pl.ds(g * 16, 16)]
            gv = gidx_v[pl.ds(g * 16, 16)]
            for l in range(16):
                uidx_s[g * 16 + l] = jnp.sum(jnp.where(lanes == l, uv, 0))
                gidx_s[g * 16 + l] = jnp.sum(jnp.where(lanes == l, gv, 0))
            return _

        lax.fori_loop(0, _BPW // 16, to_smem, 0)

        for h in range(2):
            hoff = h * half

            def chunk(c, _, hoff=hoff):
                o = c * _K
                for j in range(_K):
                    pltpu.make_async_copy(
                        utab_hbm.at[pl.ds(uidx_s[hoff + o + j], 1)],
                        urows_v.at[pl.ds(o + j, 1)], usem).start()
                    pltpu.make_async_copy(
                        gtab_hbm.at[pl.ds(gidx_s[hoff + o + j], 1)],
                        grows_v.at[pl.ds(o + j, 1)], gsem).start()
                for j in range(_K):
                    pltpu.make_async_copy(
                        utab_hbm.at[pl.ds(0, 1)],
                        urows_v.at[pl.ds(o + j, 1)], usem).wait()
                    pltpu.make_async_copy(
                        gtab_hbm.at[pl.ds(0, 1)],
                        grows_v.at[pl.ds(o + j, 1)], gsem).wait()
                return _

            lax.fori_loop(0, half // _K, chunk, 0)
            pltpu.sync_copy(urows_v, uout_hbm.at[pl.ds(base + hoff, half)])
            pltpu.sync_copy(grows_v, gout_hbm.at[pl.ds(base + hoff, half)])

    return gather2


_sc_gather_cache = []


def _sc_gather(uidx, utab, gidx, gtab):
    if not _sc_gather_cache:
        _sc_gather_cache.append(_make_sc_gather())
    return _sc_gather_cache[0](uidx, utab, gidx, gtab)


def _mlp_body(gf_ref, gl_ref, umf_ref, gmf_ref,
              gw1_ref, gb1_ref, gw2_ref, gb2_ref,
              uw1_ref, ub1_ref, uw2_ref, ub2_ref,
              dw1a_ref, dw1b_ref, db1_ref, dw2_ref, db2_ref,
              out_ref):
    f32 = jnp.float32
    g1 = jnp.maximum(
        jnp.dot(gf_ref[...], gw1_ref[...], preferred_element_type=f32) + gb1_ref[...], 0.0)
    genc = jnp.maximum(
        jnp.dot(g1, gw2_ref[...], preferred_element_type=f32) + gb2_ref[...], 0.0)
    u1 = jnp.maximum(
        jnp.dot(gl_ref[...], uw1_ref[...], preferred_element_type=f32) + ub1_ref[...], 0.0)
    uenc = jnp.maximum(
        jnp.dot(u1, uw2_ref[...], preferred_element_type=f32) + ub2_ref[...], 0.0)
    fu = umf_ref[...] + uenc
    fg = gmf_ref[...] + genc
    h = jnp.maximum(
        jnp.dot(fu, dw1a_ref[...], preferred_element_type=f32)
        + jnp.dot(fg, dw1b_ref[...], preferred_element_type=f32)
        + db1_ref[...], 0.0)
    out_ref[...] = jnp.dot(h, dw2_ref[...], preferred_element_type=f32) + db2_ref[...]


_R = 2048


def _dense(gf, gl, umf, gmf, gW1, gb1, gW2, gb2, uW1, ub1, uW2, ub2,
           dW1a, dW1b, db1, dW2, db2):
    nblk = _B // _R

    def rows(i):
        return (i, 0)

    def whole(i):
        return (0, 0)

    row_spec_feat = pl.BlockSpec((_R, gf.shape[1]), rows)
    row_spec_emb = pl.BlockSpec((_R, _EMB), rows)

    def wspec(a):
        return pl.BlockSpec(a.shape, whole)

    out = pl.pallas_call(
        _mlp_body,
        grid=(nblk,),
        in_specs=[
            row_spec_feat, row_spec_feat, row_spec_emb, row_spec_emb,
            wspec(gW1), wspec(gb1), wspec(gW2), wspec(gb2),
            wspec(uW1), wspec(ub1), wspec(uW2), wspec(ub2),
            wspec(dW1a), wspec(dW1b), wspec(db1), wspec(dW2), wspec(db2),
        ],
        out_specs=pl.BlockSpec((_R, 1), rows),
        out_shape=jax.ShapeDtypeStruct((_B, 1), jnp.float32),
    )(gf, gl, umf, gmf, gW1, gb1, gW2, gb2, uW1, ub1, uW2, ub2,
      dW1a, dW1b, db1, dW2, db2)
    return out[:, 0]


def kernel(user_input, game_input, game_features, global_features,
           user_table, game_table,
           gW1, gb1, gW2, gb2,
           uW1, ub1, uW2, ub2,
           dW1, db1, dW2, db2):
    umf, gmf = _sc_gather(user_input, user_table, game_input, game_table)
    dW1a = dW1[:_EMB]
    dW1b = dW1[_EMB:]
    return _dense(
        game_features, global_features, umf, gmf,
        gW1, gb1.reshape(1, -1), gW2, gb2.reshape(1, -1),
        uW1, ub1.reshape(1, -1), uW2, ub2.reshape(1, -1),
        dW1a, dW1b, db1.reshape(1, -1), dW2, db2.reshape(1, -1))
